# baseline (speedup 1.0000x reference)
import functools
import math

import jax
import jax.numpy as jnp
from jax import lax
from jax.experimental import pallas as pl
from jax.experimental.pallas import tpu as pltpu
from jax.experimental.pallas import tpu_sc as plsc

_NC = 80
_IOU_THR = 0.45
_HYP_CLS = 0.5
_HYP_BOX = 7.5
_EPS = 1e-7

_N = 5000
_M = 2000
_NPAD = 5120
_MPAD = 2048
_NW = 32
_PW = _NPAD // _NW
_GROUPS = _PW // 16
_CHUNKS = _MPAD // 16

_mesh = plsc.VectorSubcoreMesh(core_axis_name="c", subcore_axis_name="s")


@functools.partial(
    pl.kernel,
    mesh=_mesh,
    out_type=[
        jax.ShapeDtypeStruct((_NPAD,), jnp.float32),
        jax.ShapeDtypeStruct((_NPAD,), jnp.float32),
        jax.ShapeDtypeStruct((_NPAD,), jnp.float32),
        jax.ShapeDtypeStruct((_NPAD,), jnp.float32),
        jax.ShapeDtypeStruct((_NPAD,), jnp.float32),
        jax.ShapeDtypeStruct((_NPAD,), jnp.float32),
        jax.ShapeDtypeStruct((_NW * _MPAD,), jnp.float32),
    ],
    scratch_types=[
        pltpu.VMEM((_PW * 16,), jnp.float32),
        pltpu.VMEM((_PW * 16,), jnp.float32),
        pltpu.VMEM((_PW * 16,), jnp.float32),
        pltpu.VMEM((_PW * 16,), jnp.float32),
        pltpu.VMEM((_MPAD,), jnp.float32),
        pltpu.VMEM((_MPAD,), jnp.float32),
        pltpu.VMEM((_MPAD,), jnp.float32),
        pltpu.VMEM((_MPAD,), jnp.float32),
        pltpu.VMEM((_MPAD,), jnp.float32),
        pltpu.VMEM((_MPAD,), jnp.float32),
        pltpu.VMEM((_PW,), jnp.float32),
        pltpu.VMEM((_PW,), jnp.float32),
        pltpu.VMEM((_PW,), jnp.float32),
        pltpu.VMEM((_PW,), jnp.float32),
        pltpu.VMEM((_PW,), jnp.float32),
        pltpu.VMEM((_PW,), jnp.float32),
        pltpu.VMEM((_MPAD,), jnp.float32),
        pltpu.VMEM((16,), jnp.float32),
        pltpu.VMEM((16,), jnp.int32),
    ],
    compiler_params=pltpu.CompilerParams(needs_layout_passes=False),
)
def _sc_match(px1_h, py1_h, px2_h, py2_h, tx1_h, ty1_h, tx2_h, ty2_h,
              a2e_h, tcls_h,
              mx_o, gx1_o, gy1_o, gx2_o, gy2_o, gcls_o, matched_o,
              px1_v, py1_v, px2_v, py2_v, tx1_v, ty1_v, tx2_v, ty2_v,
              a2e_v, tcls_v,
              mxb, x1b, y1b, x2b, y2b, clb, matched_v, bfly_f, bfly_i):
    wid = lax.axis_index("s") * 2 + lax.axis_index("c")
    base = wid * _PW

    pltpu.sync_copy(px1_h.at[pl.ds(base * 16, _PW * 16)], px1_v)
    pltpu.sync_copy(py1_h.at[pl.ds(base * 16, _PW * 16)], py1_v)
    pltpu.sync_copy(px2_h.at[pl.ds(base * 16, _PW * 16)], px2_v)
    pltpu.sync_copy(py2_h.at[pl.ds(base * 16, _PW * 16)], py2_v)
    pltpu.sync_copy(tx1_h, tx1_v)
    pltpu.sync_copy(ty1_h, ty1_v)
    pltpu.sync_copy(tx2_h, tx2_v)
    pltpu.sync_copy(ty2_h, ty2_v)
    pltpu.sync_copy(a2e_h, a2e_v)
    pltpu.sync_copy(tcls_h, tcls_v)

    lane = lax.iota(jnp.int32, 16)
    zf = jnp.zeros((16,), jnp.float32)

    def _allred_f(vec, op):
        for shift in (8, 4, 2, 1):
            bfly_f[...] = vec
            vec = op(vec, plsc.load_gather(bfly_f, [lane ^ shift]))
        return vec

    def _allred_i(vec, op):
        for shift in (8, 4, 2, 1):
            bfly_i[...] = vec
            vec = op(vec, plsc.load_gather(bfly_i, [lane ^ shift]))
        return vec

    def zero_body(j, _):
        matched_v[pl.ds(j * 16, 16)] = zf
        return 0

    lax.fori_loop(0, _CHUNKS, zero_body, 0)

    def group_body(g, _):
        def pred_body(k, acc):
            accmx, accidx = acc
            off = (g * 16 + k) * 16
            x1 = px1_v[pl.ds(off, 16)]
            y1 = py1_v[pl.ds(off, 16)]
            x2 = px2_v[pl.ds(off, 16)]
            y2 = py2_v[pl.ds(off, 16)]
            a1 = (x2 - x1) * (y2 - y1)

            def chunk_body(j, c):
                mxv, agv = c
                co = j * 16
                u1 = tx1_v[pl.ds(co, 16)]
                v1 = ty1_v[pl.ds(co, 16)]
                u2 = tx2_v[pl.ds(co, 16)]
                v2 = ty2_v[pl.ds(co, 16)]
                ae = a2e_v[pl.ds(co, 16)]
                iw = jnp.maximum(jnp.minimum(x2, u2) - jnp.maximum(x1, u1),
                                 0.0)
                ih = jnp.maximum(jnp.minimum(y2, v2) - jnp.maximum(y1, v1),
                                 0.0)
                inter = iw * ih
                iou = inter / ((a1 + ae) - inter)
                upd = iou > mxv
                mxv = jnp.where(upd, iou, mxv)
                agv = jnp.where(upd, lane + co, agv)
                return (mxv, agv)

            mxv, agv = lax.fori_loop(
                0, _CHUNKS, chunk_body,
                (jnp.full((16,), -1.0, jnp.float32),
                 jnp.zeros((16,), jnp.int32)))
            m = _allred_f(mxv, jnp.maximum)
            cand = jnp.where(mxv == m, agv,
                             jnp.full((16,), 2 * _MPAD, jnp.int32))
            idx = _allred_i(cand, jnp.minimum)
            sel = lane == k
            accmx = jnp.where(sel, m, accmx)
            accidx = jnp.where(sel, idx, accidx)
            return (accmx, accidx)

        accmx, accidx = lax.fori_loop(
            0, 16, pred_body, (zf, jnp.zeros((16,), jnp.int32)))

        gx1 = plsc.load_gather(tx1_v, [accidx])
        gy1 = plsc.load_gather(ty1_v, [accidx])
        gx2 = plsc.load_gather(tx2_v, [accidx])
        gy2 = plsc.load_gather(ty2_v, [accidx])
        gcl = plsc.load_gather(tcls_v, [accidx])
        plsc.store_scatter(matched_v, [accidx],
                           jnp.ones((16,), jnp.float32),
                           mask=accmx > _IOU_THR)

        o = g * 16
        mxb[pl.ds(o, 16)] = accmx
        x1b[pl.ds(o, 16)] = gx1
        y1b[pl.ds(o, 16)] = gy1
        x2b[pl.ds(o, 16)] = gx2
        y2b[pl.ds(o, 16)] = gy2
        clb[pl.ds(o, 16)] = gcl
        return 0

    lax.fori_loop(0, _GROUPS, group_body, 0)

    pltpu.sync_copy(mxb, mx_o.at[pl.ds(base, _PW)])
    pltpu.sync_copy(x1b, gx1_o.at[pl.ds(base, _PW)])
    pltpu.sync_copy(y1b, gy1_o.at[pl.ds(base, _PW)])
    pltpu.sync_copy(x2b, gx2_o.at[pl.ds(base, _PW)])
    pltpu.sync_copy(y2b, gy2_o.at[pl.ds(base, _PW)])
    pltpu.sync_copy(clb, gcls_o.at[pl.ds(base, _PW)])
    pltpu.sync_copy(matched_v, matched_o.at[pl.ds(wid * _MPAD, _MPAD)])


def _atan(x):
    t = jnp.abs(x)
    c1 = t > 2.414213562373095
    c2 = t > 0.4142135623730951
    xr = jnp.where(c1, -1.0 / jnp.maximum(t, 1e-30),
                   jnp.where(c2, (t - 1.0) / (t + 1.0), t))
    y0 = jnp.where(c1, math.pi / 2, jnp.where(c2, math.pi / 4, 0.0))
    z = xr * xr
    p = (((8.05374449538e-2 * z - 1.38776856032e-1) * z
          + 1.99777106478e-1) * z - 3.33329491539e-1) * z * xr + xr
    return jnp.sign(x) * (y0 + p)


def _epi_body(px1_r, py1_r, px2_r, py2_r, s_r, pcls_r,
              mx_r, gx1_r, gy1_r, gx2_r, gy2_r, gcls_r, m32_r, out_ref):
    px1 = px1_r[...]
    py1 = py1_r[...]
    px2 = px2_r[...]
    py2 = py2_r[...]
    s = s_r[...]
    pcls = pcls_r[...]
    mx = mx_r[...]
    m_x1 = gx1_r[...]
    m_y1 = gy1_r[...]
    m_x2 = gx2_r[...]
    m_y2 = gy2_r[...]
    m_cls = gcls_r[...]

    keep = mx > _IOU_THR
    kf = keep.astype(jnp.float32)

    s_pos = s > 0.0
    logs = jnp.where(s_pos, jnp.log(jnp.where(s_pos, s, 1.0)), -100.0)
    log1ms = jnp.maximum(jnp.log1p(-s), -100.0)
    same = m_cls == pcls
    bce = jnp.where(keep, jnp.where(same, -logs, 100.0 - log1ms), 0.0)
    unm = jnp.where(~keep, logs, 0.0)

    w1 = px2 - px1
    h1 = py2 - py1 + _EPS
    w2 = m_x2 - m_x1
    h2 = m_y2 - m_y1 + _EPS
    inter_c = (jnp.maximum(jnp.minimum(px2, m_x2) - jnp.maximum(px1, m_x1),
                           0.0)
               * jnp.maximum(jnp.minimum(py2, m_y2) - jnp.maximum(py1, m_y1),
                             0.0))
    union_c = w1 * h1 + w2 * h2 - inter_c + _EPS
    iou_c = inter_c / union_c
    cw = jnp.maximum(px2, m_x2) - jnp.minimum(px1, m_x1)
    ch = jnp.maximum(py2, m_y2) - jnp.minimum(py1, m_y1)
    c2 = cw * cw + ch * ch + _EPS
    rho2 = ((m_x1 + m_x2 - px1 - px2) ** 2
            + (m_y1 + m_y2 - py1 - py2) ** 2) / 4.0
    datan = _atan((w2 * h1 - w1 * h2) / (h1 * h2 + w1 * w2))
    v = (4.0 / math.pi ** 2) * datan ** 2
    alpha = v / (v - iou_c + (1.0 + _EPS))
    ciou = iou_c - (rho2 / c2 + v * alpha)
    bbox = jnp.where(keep, 1.0 - ciou, 0.0)

    nk = jnp.maximum(jnp.sum(kf), 1.0)
    matched_cnt = jnp.sum(
        (jnp.max(m32_r[...], axis=0, keepdims=True) > 0.0)
        .astype(jnp.float32))
    cls_loss = (jnp.sum(bce) / (nk * float(_NC)) - jnp.sum(unm)
                + (float(_M) - matched_cnt))
    bbox_loss = jnp.sum(bbox) / nk
    total = _HYP_CLS * cls_loss + _HYP_BOX * bbox_loss
    out_ref[...] = jnp.full((8, 128), total, jnp.float32)


def kernel(preds, targets):
    pf = preds.astype(jnp.float32)
    tf = targets.astype(jnp.float32)

    preds_pad = jnp.zeros((_NPAD, 6), jnp.float32).at[:, 4].set(1.0)
    preds_pad = preds_pad.at[:_N, :].set(pf)

    px1b = jnp.repeat(preds_pad[:, 0], 16)
    py1b = jnp.repeat(preds_pad[:, 1], 16)
    px2b = jnp.repeat(preds_pad[:, 2], 16)
    py2b = jnp.repeat(preds_pad[:, 3], 16)

    tx1 = jnp.zeros((_MPAD,), jnp.float32).at[:_M].set(tf[:, 0])
    ty1 = jnp.zeros((_MPAD,), jnp.float32).at[:_M].set(tf[:, 1])
    tx2 = jnp.zeros((_MPAD,), jnp.float32).at[:_M].set(tf[:, 2])
    ty2 = jnp.zeros((_MPAD,), jnp.float32).at[:_M].set(tf[:, 3])
    tcl = jnp.zeros((_MPAD,), jnp.float32).at[:_M].set(tf[:, 4])
    a2e = jnp.full((_MPAD,), _EPS, jnp.float32).at[:_M].add(
        (tf[:, 2] - tf[:, 0]) * (tf[:, 3] - tf[:, 1]))

    mx, gx1, gy1, gx2, gy2, gcl, matched = _sc_match(
        px1b, py1b, px2b, py2b, tx1, ty1, tx2, ty2, a2e, tcl)

    rs = (_NPAD // 128, 128)
    out = pl.pallas_call(
        _epi_body,
        out_specs=pl.BlockSpec((8, 128), lambda: (0, 0)),
        out_shape=jax.ShapeDtypeStruct((8, 128), jnp.float32),
    )(
        preds_pad[:, 0].reshape(rs), preds_pad[:, 1].reshape(rs),
        preds_pad[:, 2].reshape(rs), preds_pad[:, 3].reshape(rs),
        preds_pad[:, 4].reshape(rs), preds_pad[:, 5].reshape(rs),
        mx.reshape(rs), gx1.reshape(rs), gy1.reshape(rs),
        gx2.reshape(rs), gy2.reshape(rs), gcl.reshape(rs),
        matched.reshape(_NW, _MPAD),
    )
    return out[0, 0]


# speedup vs baseline: 1.2804x; 1.2804x over previous
import functools
import math

import jax
import jax.numpy as jnp
from jax import lax
from jax.experimental import pallas as pl
from jax.experimental.pallas import tpu as pltpu
from jax.experimental.pallas import tpu_sc as plsc

_NC = 80
_IOU_THR = 0.45
_HYP_CLS = 0.5
_HYP_BOX = 7.5
_EPS = 1e-7

_N = 5000
_M = 2000
_NPAD = 5120
_MPAD = 2048

_S = 2048
_NW = 32
_PW = _S // _NW
_GROUPS = _PW // 16
_CHUNKS = _MPAD // 16

_NT = _NPAD - _S
_RT = 1024

_mesh = plsc.VectorSubcoreMesh(core_axis_name="c", subcore_axis_name="s")


@functools.partial(
    pl.kernel,
    mesh=_mesh,
    out_type=[
        jax.ShapeDtypeStruct((_S,), jnp.float32),
        jax.ShapeDtypeStruct((_S,), jnp.float32),
        jax.ShapeDtypeStruct((_S,), jnp.float32),
        jax.ShapeDtypeStruct((_S,), jnp.float32),
        jax.ShapeDtypeStruct((_S,), jnp.float32),
        jax.ShapeDtypeStruct((_S,), jnp.float32),
        jax.ShapeDtypeStruct((_NW * _MPAD,), jnp.float32),
    ],
    scratch_types=[
        pltpu.VMEM((_PW * 16,), jnp.float32),
        pltpu.VMEM((_PW * 16,), jnp.float32),
        pltpu.VMEM((_PW * 16,), jnp.float32),
        pltpu.VMEM((_PW * 16,), jnp.float32),
        pltpu.VMEM((_MPAD,), jnp.float32),
        pltpu.VMEM((_MPAD,), jnp.float32),
        pltpu.VMEM((_MPAD,), jnp.float32),
        pltpu.VMEM((_MPAD,), jnp.float32),
        pltpu.VMEM((_MPAD,), jnp.float32),
        pltpu.VMEM((_MPAD,), jnp.float32),
        pltpu.VMEM((_PW,), jnp.float32),
        pltpu.VMEM((_PW,), jnp.float32),
        pltpu.VMEM((_PW,), jnp.float32),
        pltpu.VMEM((_PW,), jnp.float32),
        pltpu.VMEM((_PW,), jnp.float32),
        pltpu.VMEM((_PW,), jnp.float32),
        pltpu.VMEM((_MPAD,), jnp.float32),
        pltpu.VMEM((16,), jnp.float32),
        pltpu.VMEM((16,), jnp.int32),
    ],
    compiler_params=pltpu.CompilerParams(needs_layout_passes=False),
)
def _sc_match(px1_h, py1_h, px2_h, py2_h, tx1_h, ty1_h, tx2_h, ty2_h,
              a2e_h, tcls_h,
              mx_o, gx1_o, gy1_o, gx2_o, gy2_o, gcls_o, matched_o,
              px1_v, py1_v, px2_v, py2_v, tx1_v, ty1_v, tx2_v, ty2_v,
              a2e_v, tcls_v,
              mxb, x1b, y1b, x2b, y2b, clb, matched_v, bfly_f, bfly_i):
    wid = lax.axis_index("s") * 2 + lax.axis_index("c")
    base = wid * _PW

    pltpu.sync_copy(px1_h.at[pl.ds(base * 16, _PW * 16)], px1_v)
    pltpu.sync_copy(py1_h.at[pl.ds(base * 16, _PW * 16)], py1_v)
    pltpu.sync_copy(px2_h.at[pl.ds(base * 16, _PW * 16)], px2_v)
    pltpu.sync_copy(py2_h.at[pl.ds(base * 16, _PW * 16)], py2_v)
    pltpu.sync_copy(tx1_h, tx1_v)
    pltpu.sync_copy(ty1_h, ty1_v)
    pltpu.sync_copy(tx2_h, tx2_v)
    pltpu.sync_copy(ty2_h, ty2_v)
    pltpu.sync_copy(a2e_h, a2e_v)
    pltpu.sync_copy(tcls_h, tcls_v)

    lane = lax.iota(jnp.int32, 16)
    zf = jnp.zeros((16,), jnp.float32)

    def _allred_f(vec, op):
        for shift in (8, 4, 2, 1):
            bfly_f[...] = vec
            vec = op(vec, plsc.load_gather(bfly_f, [lane ^ shift]))
        return vec

    def _allred_i(vec, op):
        for shift in (8, 4, 2, 1):
            bfly_i[...] = vec
            vec = op(vec, plsc.load_gather(bfly_i, [lane ^ shift]))
        return vec

    def zero_body(j, _):
        matched_v[pl.ds(j * 16, 16)] = zf
        return 0

    lax.fori_loop(0, _CHUNKS, zero_body, 0)

    def group_body(g, _):
        def pred_body(k, acc):
            accmx, accidx = acc
            off = (g * 16 + k) * 16
            x1 = px1_v[pl.ds(off, 16)]
            y1 = py1_v[pl.ds(off, 16)]
            x2 = px2_v[pl.ds(off, 16)]
            y2 = py2_v[pl.ds(off, 16)]
            a1 = (x2 - x1) * (y2 - y1)

            def chunk_body(j, c):
                mxv, agv = c
                co = j * 16
                u1 = tx1_v[pl.ds(co, 16)]
                v1 = ty1_v[pl.ds(co, 16)]
                u2 = tx2_v[pl.ds(co, 16)]
                v2 = ty2_v[pl.ds(co, 16)]
                ae = a2e_v[pl.ds(co, 16)]
                iw = jnp.maximum(jnp.minimum(x2, u2) - jnp.maximum(x1, u1),
                                 0.0)
                ih = jnp.maximum(jnp.minimum(y2, v2) - jnp.maximum(y1, v1),
                                 0.0)
                inter = iw * ih
                iou = inter / ((a1 + ae) - inter)
                upd = iou > mxv
                mxv = jnp.where(upd, iou, mxv)
                agv = jnp.where(upd, lane + co, agv)
                return (mxv, agv)

            mxv, agv = lax.fori_loop(
                0, _CHUNKS, chunk_body,
                (jnp.full((16,), -1.0, jnp.float32),
                 jnp.zeros((16,), jnp.int32)))
            m = _allred_f(mxv, jnp.maximum)
            cand = jnp.where(mxv == m, agv,
                             jnp.full((16,), 2 * _MPAD, jnp.int32))
            idx = _allred_i(cand, jnp.minimum)
            sel = lane == k
            accmx = jnp.where(sel, m, accmx)
            accidx = jnp.where(sel, idx, accidx)
            return (accmx, accidx)

        accmx, accidx = lax.fori_loop(
            0, 16, pred_body, (zf, jnp.zeros((16,), jnp.int32)))

        gx1 = plsc.load_gather(tx1_v, [accidx])
        gy1 = plsc.load_gather(ty1_v, [accidx])
        gx2 = plsc.load_gather(tx2_v, [accidx])
        gy2 = plsc.load_gather(ty2_v, [accidx])
        gcl = plsc.load_gather(tcls_v, [accidx])
        plsc.store_scatter(matched_v, [accidx],
                           jnp.ones((16,), jnp.float32),
                           mask=accmx > _IOU_THR)

        o = g * 16
        mxb[pl.ds(o, 16)] = accmx
        x1b[pl.ds(o, 16)] = gx1
        y1b[pl.ds(o, 16)] = gy1
        x2b[pl.ds(o, 16)] = gx2
        y2b[pl.ds(o, 16)] = gy2
        clb[pl.ds(o, 16)] = gcl
        return 0

    lax.fori_loop(0, _GROUPS, group_body, 0)

    pltpu.sync_copy(mxb, mx_o.at[pl.ds(base, _PW)])
    pltpu.sync_copy(x1b, gx1_o.at[pl.ds(base, _PW)])
    pltpu.sync_copy(y1b, gy1_o.at[pl.ds(base, _PW)])
    pltpu.sync_copy(x2b, gx2_o.at[pl.ds(base, _PW)])
    pltpu.sync_copy(y2b, gy2_o.at[pl.ds(base, _PW)])
    pltpu.sync_copy(clb, gcls_o.at[pl.ds(base, _PW)])
    pltpu.sync_copy(matched_v, matched_o.at[pl.ds(wid * _MPAD, _MPAD)])


def _atan(x):
    t = jnp.abs(x)
    c1 = t > 2.414213562373095
    c2 = t > 0.4142135623730951
    xr = jnp.where(c1, -1.0 / jnp.maximum(t, 1e-30),
                   jnp.where(c2, (t - 1.0) / (t + 1.0), t))
    y0 = jnp.where(c1, math.pi / 2, jnp.where(c2, math.pi / 4, 0.0))
    z = xr * xr
    p = (((8.05374449538e-2 * z - 1.38776856032e-1) * z
          + 1.99777106478e-1) * z - 3.33329491539e-1) * z * xr + xr
    return jnp.sign(x) * (y0 + p)


def _row_losses(px1, py1, px2, py2, s, pcls, keep,
                m_x1, m_y1, m_x2, m_y2, m_cls):
    s_pos = s > 0.0
    logs = jnp.where(s_pos, jnp.log(jnp.where(s_pos, s, 1.0)), -100.0)
    log1ms = jnp.maximum(jnp.log1p(-s), -100.0)
    same = m_cls == pcls
    bce = jnp.where(keep, jnp.where(same, -logs, 100.0 - log1ms), 0.0)
    unm = jnp.where(~keep, logs, 0.0)

    w1 = px2 - px1
    h1 = py2 - py1 + _EPS
    w2 = m_x2 - m_x1
    h2 = m_y2 - m_y1 + _EPS
    inter_c = (jnp.maximum(jnp.minimum(px2, m_x2) - jnp.maximum(px1, m_x1),
                           0.0)
               * jnp.maximum(jnp.minimum(py2, m_y2) - jnp.maximum(py1, m_y1),
                             0.0))
    union_c = w1 * h1 + w2 * h2 - inter_c + _EPS
    iou_c = inter_c / union_c
    cw = jnp.maximum(px2, m_x2) - jnp.minimum(px1, m_x1)
    ch = jnp.maximum(py2, m_y2) - jnp.minimum(py1, m_y1)
    c2 = cw * cw + ch * ch + _EPS
    rho2 = ((m_x1 + m_x2 - px1 - px2) ** 2
            + (m_y1 + m_y2 - py1 - py2) ** 2) / 4.0
    datan = _atan((w2 * h1 - w1 * h2) / (h1 * h2 + w1 * w2))
    v = (4.0 / math.pi ** 2) * datan ** 2
    alpha = v / (v - iou_c + (1.0 + _EPS))
    ciou = iou_c - (rho2 / c2 + v * alpha)
    bbox = jnp.where(keep, 1.0 - ciou, 0.0)
    return bce, unm, bbox


def _tc_body(preds_ref, tgt_ref, tgt2_ref, scal_ref, colm_ref, sacc_ref):
    i = pl.program_id(0)
    nt = pl.num_programs(0)

    @pl.when(i == 0)
    def _init():
        colm_ref[0:1, :] = jnp.zeros((1, _MPAD), jnp.float32)
        for k in range(4):
            sacc_ref[k] = 0.0

    P = preds_ref[...]
    px1 = P[:, 0:1]
    py1 = P[:, 1:2]
    px2 = P[:, 2:3]
    py2 = P[:, 3:4]
    s = P[:, 4:5]
    pcls = P[:, 5:6]

    T = tgt_ref[...]
    tx1 = T[0:1, :]
    ty1 = T[1:2, :]
    tx2 = T[2:3, :]
    ty2 = T[3:4, :]
    a2e = T[5:6, :]

    col = jax.lax.broadcasted_iota(jnp.int32, (1, _MPAD), 1).astype(
        jnp.float32)

    a1 = (px2 - px1) * (py2 - py1)
    iw = jnp.maximum(jnp.minimum(px2, tx2) - jnp.maximum(px1, tx1), 0.0)
    ih = jnp.maximum(jnp.minimum(py2, ty2) - jnp.maximum(py1, ty1), 0.0)
    inter = iw * ih
    iou = inter / ((a1 + a2e) - inter)

    mx = jnp.max(iou, axis=1, keepdims=True)
    idxv = jnp.min(jnp.where(iou == mx, col, float(2 * _MPAD)),
                   axis=1, keepdims=True)
    ohf = (col == idxv).astype(jnp.float32)

    keep = mx > _IOU_THR
    kf = keep.astype(jnp.float32)

    m = jax.lax.dot_general(ohf, tgt2_ref[...], (((1,), (0,)), ((), ())),
                            preferred_element_type=jnp.float32)

    bce, unm, bbox = _row_losses(px1, py1, px2, py2, s, pcls, keep,
                                 m[:, 0:1], m[:, 1:2], m[:, 2:3], m[:, 3:4],
                                 m[:, 4:5])

    sacc_ref[0] += jnp.sum(kf)
    sacc_ref[1] += jnp.sum(bce)
    sacc_ref[2] += jnp.sum(unm)
    sacc_ref[3] += jnp.sum(bbox)
    colm = jax.lax.dot_general(kf, ohf, (((0,), (0,)), ((), ())),
                               preferred_element_type=jnp.float32)
    colm_ref[0:1, :] += colm

    @pl.when(i == nt - 1)
    def _fin():
        r = jax.lax.broadcasted_iota(jnp.int32, (8, 128), 0)
        c = jax.lax.broadcasted_iota(jnp.int32, (8, 128), 1)
        vals = jnp.where(
            c == 0, sacc_ref[0],
            jnp.where(c == 1, sacc_ref[1],
                      jnp.where(c == 2, sacc_ref[2], sacc_ref[3])))
        scal_ref[...] = jnp.where(r == 0, vals, 0.0)


def _epi_body(px1_r, py1_r, px2_r, py2_r, s_r, pcls_r,
              mx_r, gx1_r, gy1_r, gx2_r, gy2_r, gcls_r,
              m32_r, tscal_r, tcolm_r, out_ref):
    keep = mx_r[...] > _IOU_THR
    bce, unm, bbox = _row_losses(
        px1_r[...], py1_r[...], px2_r[...], py2_r[...], s_r[...], pcls_r[...],
        keep, gx1_r[...], gy1_r[...], gx2_r[...], gy2_r[...], gcls_r[...])

    ts = tscal_r[...]
    nk = jnp.maximum(jnp.sum(keep.astype(jnp.float32)) + ts[0, 0], 1.0)
    bce_sum = jnp.sum(bce) + ts[0, 1]
    unm_sum = jnp.sum(unm) + ts[0, 2]
    bbox_sum = jnp.sum(bbox) + ts[0, 3]

    matched = (jnp.max(m32_r[...], axis=0, keepdims=True)
               + tcolm_r[0:1, :]) > 0.0
    matched_cnt = jnp.sum(matched.astype(jnp.float32))

    cls_loss = (bce_sum / (nk * float(_NC)) - unm_sum
                + (float(_M) - matched_cnt))
    total = _HYP_CLS * cls_loss + _HYP_BOX * (bbox_sum / nk)
    out_ref[...] = jnp.full((8, 128), total, jnp.float32)


def kernel(preds, targets):
    pf = preds.astype(jnp.float32)
    tf = targets.astype(jnp.float32)

    preds_pad = jnp.zeros((_NPAD, 8), jnp.float32).at[:, 4].set(1.0)
    preds_pad = preds_pad.at[:_N, :6].set(pf)

    px1b = jnp.repeat(preds_pad[:_S, 0], 16)
    py1b = jnp.repeat(preds_pad[:_S, 1], 16)
    px2b = jnp.repeat(preds_pad[:_S, 2], 16)
    py2b = jnp.repeat(preds_pad[:_S, 3], 16)

    tx1 = jnp.zeros((_MPAD,), jnp.float32).at[:_M].set(tf[:, 0])
    ty1 = jnp.zeros((_MPAD,), jnp.float32).at[:_M].set(tf[:, 1])
    tx2 = jnp.zeros((_MPAD,), jnp.float32).at[:_M].set(tf[:, 2])
    ty2 = jnp.zeros((_MPAD,), jnp.float32).at[:_M].set(tf[:, 3])
    tcl = jnp.zeros((_MPAD,), jnp.float32).at[:_M].set(tf[:, 4])
    a2 = (tf[:, 2] - tf[:, 0]) * (tf[:, 3] - tf[:, 1])
    a2e = jnp.full((_MPAD,), _EPS, jnp.float32).at[:_M].add(a2)

    mx, gx1, gy1, gx2, gy2, gcl, matched = _sc_match(
        px1b, py1b, px2b, py2b, tx1, ty1, tx2, ty2, a2e, tcl)

    tgt_t = jnp.zeros((8, _MPAD), jnp.float32)
    tgt_t = tgt_t.at[:5, :_M].set(tf.T)
    tgt_t = tgt_t.at[5, :].set(_EPS)
    tgt_t = tgt_t.at[5, :_M].add(a2)
    tgt2 = jnp.zeros((_MPAD, 8), jnp.float32)
    tgt2 = tgt2.at[:_M, :5].set(tf)

    tscal, tcolm = pl.pallas_call(
        _tc_body,
        grid=(_NT // _RT,),
        in_specs=[
            pl.BlockSpec((_RT, 8), lambda i: (i, 0)),
            pl.BlockSpec((8, _MPAD), lambda i: (0, 0)),
            pl.BlockSpec((_MPAD, 8), lambda i: (0, 0)),
        ],
        out_specs=[
            pl.BlockSpec((8, 128), lambda i: (0, 0)),
            pl.BlockSpec((8, _MPAD), lambda i: (0, 0)),
        ],
        out_shape=[
            jax.ShapeDtypeStruct((8, 128), jnp.float32),
            jax.ShapeDtypeStruct((8, _MPAD), jnp.float32),
        ],
        scratch_shapes=[pltpu.SMEM((8,), jnp.float32)],
        compiler_params=pltpu.CompilerParams(
            dimension_semantics=("arbitrary",),
        ),
    )(preds_pad[_S:], tgt_t, tgt2)

    rs = (_S // 128, 128)
    out = pl.pallas_call(
        _epi_body,
        out_specs=pl.BlockSpec((8, 128), lambda: (0, 0)),
        out_shape=jax.ShapeDtypeStruct((8, 128), jnp.float32),
    )(
        preds_pad[:_S, 0].reshape(rs), preds_pad[:_S, 1].reshape(rs),
        preds_pad[:_S, 2].reshape(rs), preds_pad[:_S, 3].reshape(rs),
        preds_pad[:_S, 4].reshape(rs), preds_pad[:_S, 5].reshape(rs),
        mx.reshape(rs), gx1.reshape(rs), gy1.reshape(rs),
        gx2.reshape(rs), gy2.reshape(rs), gcl.reshape(rs),
        matched.reshape(_NW, _MPAD), tscal, tcolm,
    )
    return out[0, 0]


# speedup vs baseline: 2.0496x; 1.6008x over previous
import math

import jax
import jax.numpy as jnp
from jax.experimental import pallas as pl
from jax.experimental.pallas import tpu as pltpu

_NC = 80
_IOU_THR = 0.45
_HYP_CLS = 0.5
_HYP_BOX = 7.5
_EPS = 1e-7

_N = 5000
_M = 2000
_R = 1000
_NPAD = 5000
_MPAD = 2048


def _atan(x):
    t = jnp.abs(x)
    c1 = t > 2.414213562373095
    c2 = t > 0.4142135623730951
    xr = jnp.where(c1, -1.0 / jnp.maximum(t, 1e-30),
                   jnp.where(c2, (t - 1.0) / (t + 1.0), t))
    y0 = jnp.where(c1, math.pi / 2, jnp.where(c2, math.pi / 4, 0.0))
    z = xr * xr
    p = (((8.05374449538e-2 * z - 1.38776856032e-1) * z
          + 1.99777106478e-1) * z - 3.33329491539e-1) * z * xr + xr
    return jnp.sign(x) * (y0 + p)


def _body(preds_ref, tgt_ref, tgt2_ref, out_ref, macc_ref, sacc_ref):
    i = pl.program_id(0)
    nt = pl.num_programs(0)

    @pl.when(i == 0)
    def _init():
        macc_ref[0:1, :] = jnp.zeros((1, _MPAD), jnp.float32)
        for k in range(4):
            sacc_ref[k] = 0.0

    P = preds_ref[...]
    px1 = P[:, 0:1]
    py1 = P[:, 1:2]
    px2 = P[:, 2:3]
    py2 = P[:, 3:4]
    s = P[:, 4:5]
    pcls = P[:, 5:6]

    T = tgt_ref[...]
    tx1 = T[0:1, :]
    ty1 = T[1:2, :]
    tx2 = T[2:3, :]
    ty2 = T[3:4, :]
    a2e = T[5:6, :]

    col = jax.lax.broadcasted_iota(jnp.int32, (1, _MPAD), 1).astype(jnp.float32)

    a1 = (px2 - px1) * (py2 - py1)
    iw = jnp.maximum(jnp.minimum(px2, tx2) - jnp.maximum(px1, tx1), 0.0)
    ih = jnp.maximum(jnp.minimum(py2, ty2) - jnp.maximum(py1, ty1), 0.0)
    inter = iw * ih
    iou = inter / ((a1 + a2e) - inter)

    mx = jnp.max(iou, axis=1, keepdims=True)
    idxv = jnp.min(jnp.where(iou == mx, col, float(2 * _MPAD)),
                   axis=1, keepdims=True)
    ohf = (col == idxv).astype(jnp.float32)

    keep = mx > _IOU_THR
    kf = keep.astype(jnp.float32)

    m = jax.lax.dot_general(ohf, tgt2_ref[...], (((1,), (0,)), ((), ())),
                            preferred_element_type=jnp.float32)
    m_x1 = m[:, 0:1]
    m_y1 = m[:, 1:2]
    m_x2 = m[:, 2:3]
    m_y2 = m[:, 3:4]
    m_cls = m[:, 4:5]

    s_pos = s > 0.0
    logs = jnp.where(s_pos, jnp.log(jnp.where(s_pos, s, 1.0)), -100.0)
    log1ms = jnp.maximum(jnp.log1p(-s), -100.0)
    same = m_cls == pcls
    bce_row = jnp.where(keep, jnp.where(same, -logs, 100.0 - log1ms), 0.0)
    unm_row = jnp.where(~keep, logs, 0.0)

    w1 = px2 - px1
    h1 = py2 - py1 + _EPS
    w2 = m_x2 - m_x1
    h2 = m_y2 - m_y1 + _EPS
    inter_c = (jnp.maximum(jnp.minimum(px2, m_x2) - jnp.maximum(px1, m_x1), 0.0)
               * jnp.maximum(jnp.minimum(py2, m_y2) - jnp.maximum(py1, m_y1), 0.0))
    union_c = w1 * h1 + w2 * h2 - inter_c + _EPS
    iou_c = inter_c / union_c
    cw = jnp.maximum(px2, m_x2) - jnp.minimum(px1, m_x1)
    ch = jnp.maximum(py2, m_y2) - jnp.minimum(py1, m_y1)
    c2 = cw * cw + ch * ch + _EPS
    rho2 = ((m_x1 + m_x2 - px1 - px2) ** 2 + (m_y1 + m_y2 - py1 - py2) ** 2) / 4.0
    datan = _atan((w2 * h1 - w1 * h2) / (h1 * h2 + w1 * w2))
    v = (4.0 / math.pi ** 2) * datan ** 2
    alpha = v / (v - iou_c + (1.0 + _EPS))
    ciou = iou_c - (rho2 / c2 + v * alpha)
    bbox_row = jnp.where(keep, 1.0 - ciou, 0.0)

    sacc_ref[0] += jnp.sum(kf)
    sacc_ref[1] += jnp.sum(bce_row)
    sacc_ref[2] += jnp.sum(unm_row)
    sacc_ref[3] += jnp.sum(bbox_row)
    colm = jax.lax.dot_general(kf, ohf, (((0,), (0,)), ((), ())),
                               preferred_element_type=jnp.float32)
    macc_ref[0:1, :] += colm

    @pl.when(i == nt - 1)
    def _fin():
        nk = jnp.maximum(sacc_ref[0], 1.0)
        matched_cnt = jnp.sum((macc_ref[0:1, :] > 0.0).astype(jnp.float32))
        cls_loss = (sacc_ref[1] / (nk * float(_NC)) - sacc_ref[2]
                    + (float(_M) - matched_cnt))
        bbox_loss = sacc_ref[3] / nk
        total = _HYP_CLS * cls_loss + _HYP_BOX * bbox_loss
        out_ref[...] = jnp.full((8, 128), total, jnp.float32)


def kernel(preds, targets):
    preds_pad = jnp.zeros((_NPAD, 8), jnp.float32)
    preds_pad = preds_pad.at[:_N, :6].set(preds.astype(jnp.float32))
    tf = targets.astype(jnp.float32)
    tgt_t = jnp.zeros((8, _MPAD), jnp.float32)
    tgt_t = tgt_t.at[:5, :_M].set(tf.T)
    a2 = (tf[:, 2] - tf[:, 0]) * (tf[:, 3] - tf[:, 1])
    tgt_t = tgt_t.at[5, :].set(_EPS)
    tgt_t = tgt_t.at[5, :_M].add(a2)
    tgt2 = jnp.zeros((_MPAD, 8), jnp.float32)
    tgt2 = tgt2.at[:_M, :5].set(tf)

    nt = _NPAD // _R
    out = pl.pallas_call(
        _body,
        grid=(nt,),
        in_specs=[
            pl.BlockSpec((_R, 8), lambda i: (i, 0)),
            pl.BlockSpec((8, _MPAD), lambda i: (0, 0)),
            pl.BlockSpec((_MPAD, 8), lambda i: (0, 0)),
        ],
        out_specs=pl.BlockSpec((8, 128), lambda i: (0, 0)),
        out_shape=jax.ShapeDtypeStruct((8, 128), jnp.float32),
        scratch_shapes=[
            pltpu.VMEM((8, _MPAD), jnp.float32),
            pltpu.SMEM((8,), jnp.float32),
        ],
        compiler_params=pltpu.CompilerParams(
            dimension_semantics=("arbitrary",),
        ),
    )(preds_pad, tgt_t, tgt2)
    return out[0, 0]
